# Initial kernel scaffold; baseline (speedup 1.0000x reference)
import jax
import jax.numpy as jnp
from jax.experimental import pallas as pl


def kernel(inputs, edge_index, adj_high, W_self1, W_neigh1, b1, W_self2, W_neigh2, b2):
    raise NotImplementedError("write your pallas kernel here")


# speedup vs baseline: 3.3345x; 3.3345x over previous
import functools

import jax
import jax.numpy as jnp
from jax import lax
from jax.experimental import pallas as pl
from jax.experimental.pallas import tpu as pltpu
from jax.experimental.pallas import tpu_sc as plsc

N_NODES = 10000
N_EDGES = 320000
NC = 2
NS = 16
LANES = 16

EPC = N_EDGES // NC
EPT = EPC // NS
DEG_EPT = N_EDGES // NS
K = 80
AGG_CHUNKS = EPT // K
DEG_CHUNKS = DEG_EPT // K
RCH = 200
NCH_TOT = N_NODES // RCH
DEGW = 16

BLK = 1000


HALF = 64
N_FULL = NCH_TOT // NS


def _rr_chunks(s, fn):
  for m in range(N_FULL + 1):
    cid = s + m * NS
    if m < N_FULL:
      fn(cid)
    else:
      pl.when(cid < NCH_TOT)(lambda: fn(cid))


def _scale_rows(buf_v, inv_v):
  def body(r, carry):
    inv = inv_v[r, :]
    for q in range(HALF // LANES):
      sl = pl.ds(q * LANES, LANES)
      buf_v[r, sl] = buf_v[r, sl] * inv
    return carry
  lax.fori_loop(0, RCH, body, 0)


def _agg_pass(p_hbm, src_hbm, dst_hbm, src_v, dst_v, rows_v, acc_s, sem, c, s):
  def body(i, carry):
    base = c * EPC + s * EPT + i * K
    pltpu.sync_copy(src_hbm.at[pl.ds(base, K)], src_v)
    pltpu.sync_copy(dst_hbm.at[pl.ds(base, K)], dst_v)
    pltpu.async_copy(p_hbm.at[src_v], rows_v, sem).wait()
    pltpu.sync_copy(rows_v, acc_s.at[dst_v], add=True)
    return carry
  lax.fori_loop(0, AGG_CHUNKS, body, 0)


def _make_seg1():
  mesh = plsc.VectorSubcoreMesh(core_axis_name="c", subcore_axis_name="s")

  @functools.partial(
      pl.kernel,
      mesh=mesh,
      compiler_params=pltpu.CompilerParams(use_tc_tiling_on_sc=False),
      out_type=[
          jax.ShapeDtypeStruct((NC * N_NODES, HALF), jnp.float32),
          jax.ShapeDtypeStruct((NC * N_NODES, HALF), jnp.float32),
          jax.ShapeDtypeStruct((N_NODES, DEGW), jnp.float32),
      ],
      scratch_types=[
          pltpu.VMEM((K,), jnp.int32),
          pltpu.VMEM((K,), jnp.int32),
          pltpu.VMEM((K, HALF), jnp.float32),
          pltpu.VMEM((K, DEGW), jnp.float32),
          pltpu.VMEM((RCH, HALF), jnp.float32),
          pltpu.VMEM((RCH, HALF), jnp.float32),
          pltpu.VMEM((RCH, DEGW), jnp.float32),
          pltpu.VMEM((RCH, DEGW), jnp.float32),
          pltpu.VMEM_SHARED((N_NODES, HALF), jnp.float32),
          pltpu.VMEM_SHARED((N_NODES, DEGW), jnp.float32),
          pltpu.SemaphoreType.DMA,
      ],
  )
  def seg(pa_hbm, pb_hbm, src_hbm, dst_hbm, zrow_hbm, zdeg_hbm, ones_hbm,
          outa_hbm, outb_hbm, inv_hbm,
          src_v, dst_v, rows_v, ones_v, buf_v, zbuf_v, deg_v, zdeg_v,
          acc_s, deg_s, sem):
    c = lax.axis_index("c")
    s = lax.axis_index("s")

    pltpu.sync_copy(zrow_hbm, zbuf_v)
    pltpu.sync_copy(zdeg_hbm, zdeg_v)
    pltpu.sync_copy(ones_hbm, ones_v)

    def zero_chunk(cid):
      pltpu.sync_copy(zbuf_v, acc_s.at[pl.ds(cid * RCH, RCH)])
      pltpu.sync_copy(zdeg_v, deg_s.at[pl.ds(cid * RCH, RCH)])
    _rr_chunks(s, zero_chunk)
    plsc.subcore_barrier()

    def deg_body(i, carry):
      base = s * DEG_EPT + i * K
      pltpu.sync_copy(dst_hbm.at[pl.ds(base, K)], dst_v)
      pltpu.sync_copy(ones_v, deg_s.at[dst_v], add=True)
      return carry
    lax.fori_loop(0, DEG_CHUNKS, deg_body, 0)

    _agg_pass(pa_hbm, src_hbm, dst_hbm, src_v, dst_v, rows_v, acc_s, sem, c, s)
    plsc.subcore_barrier()

    def drain_a(cid):
      r0 = cid * RCH
      pltpu.sync_copy(acc_s.at[pl.ds(r0, RCH)], buf_v)
      pltpu.sync_copy(zbuf_v, acc_s.at[pl.ds(r0, RCH)])
      pltpu.sync_copy(deg_s.at[pl.ds(r0, RCH)], deg_v)

      def inv_body(r, carry):
        deg_v[r, :] = 1.0 / jnp.maximum(deg_v[r, :], 1.0)
        return carry
      lax.fori_loop(0, RCH, inv_body, 0)
      pltpu.sync_copy(deg_v, inv_hbm.at[pl.ds(r0, RCH)])
      _scale_rows(buf_v, deg_v)
      pltpu.sync_copy(buf_v, outa_hbm.at[pl.ds(c * N_NODES + r0, RCH)])
    _rr_chunks(s, drain_a)
    plsc.subcore_barrier()

    _agg_pass(pb_hbm, src_hbm, dst_hbm, src_v, dst_v, rows_v, acc_s, sem, c, s)
    plsc.subcore_barrier()

    def drain_b(cid):
      r0 = cid * RCH
      pltpu.sync_copy(acc_s.at[pl.ds(r0, RCH)], buf_v)
      pltpu.sync_copy(inv_hbm.at[pl.ds(r0, RCH)], deg_v)
      _scale_rows(buf_v, deg_v)
      pltpu.sync_copy(buf_v, outb_hbm.at[pl.ds(c * N_NODES + r0, RCH)])
    _rr_chunks(s, drain_b)

  return seg


def _make_seg2():
  mesh = plsc.VectorSubcoreMesh(core_axis_name="c", subcore_axis_name="s")

  @functools.partial(
      pl.kernel,
      mesh=mesh,
      compiler_params=pltpu.CompilerParams(use_tc_tiling_on_sc=False),
      out_type=jax.ShapeDtypeStruct((NC * N_NODES, HALF), jnp.float32),
      scratch_types=[
          pltpu.VMEM((K,), jnp.int32),
          pltpu.VMEM((K,), jnp.int32),
          pltpu.VMEM((K, HALF), jnp.float32),
          pltpu.VMEM((RCH, HALF), jnp.float32),
          pltpu.VMEM((RCH, HALF), jnp.float32),
          pltpu.VMEM((RCH, DEGW), jnp.float32),
          pltpu.VMEM_SHARED((N_NODES, HALF), jnp.float32),
          pltpu.SemaphoreType.DMA,
      ],
  )
  def seg(p_hbm, src_hbm, dst_hbm, zrow_hbm, inv_hbm, out_hbm,
          src_v, dst_v, rows_v, buf_v, zbuf_v, deg_v, acc_s, sem):
    c = lax.axis_index("c")
    s = lax.axis_index("s")

    pltpu.sync_copy(zrow_hbm, zbuf_v)

    def zero_chunk(cid):
      pltpu.sync_copy(zbuf_v, acc_s.at[pl.ds(cid * RCH, RCH)])
    _rr_chunks(s, zero_chunk)
    plsc.subcore_barrier()

    _agg_pass(p_hbm, src_hbm, dst_hbm, src_v, dst_v, rows_v, acc_s, sem, c, s)
    plsc.subcore_barrier()

    def drain_chunk(cid):
      r0 = cid * RCH
      pltpu.sync_copy(acc_s.at[pl.ds(r0, RCH)], buf_v)
      pltpu.sync_copy(inv_hbm.at[pl.ds(r0, RCH)], deg_v)
      _scale_rows(buf_v, deg_v)
      pltpu.sync_copy(buf_v, out_hbm.at[pl.ds(c * N_NODES + r0, RCH)])
    _rr_chunks(s, drain_chunk)

  return seg


def _tc_proj2(x, ws, wn, b, dout):
  n, din = x.shape

  def body(x_ref, ws_ref, wn_ref, b_ref, s_ref, p_ref):
    xb = x_ref[...]
    s_ref[...] = jnp.dot(xb, ws_ref[...],
                         preferred_element_type=jnp.float32) + b_ref[...]
    p_ref[...] = jnp.dot(xb, wn_ref[...], preferred_element_type=jnp.float32)

  return pl.pallas_call(
      body,
      grid=(n // BLK,),
      in_specs=[
          pl.BlockSpec((BLK, din), lambda i: (i, 0)),
          pl.BlockSpec((din, dout), lambda i: (0, 0)),
          pl.BlockSpec((din, dout), lambda i: (0, 0)),
          pl.BlockSpec((1, dout), lambda i: (0, 0)),
      ],
      out_specs=[
          pl.BlockSpec((BLK, dout), lambda i: (i, 0)),
          pl.BlockSpec((BLK, dout), lambda i: (i, 0)),
      ],
      out_shape=[
          jax.ShapeDtypeStruct((n, dout), jnp.float32),
          jax.ShapeDtypeStruct((n, dout), jnp.float32),
      ],
  )(x, ws, wn, b)


def _tc_relu_proj2(s1, aa0, aa1, ab0, ab1, ws, wn, b, dout):
  n, din = s1.shape

  def body(s1_ref, aa0_ref, aa1_ref, ab0_ref, ab1_ref, ws_ref, wn_ref, b_ref,
           s_ref, p_ref):
    agg = jnp.concatenate(
        [aa0_ref[...] + aa1_ref[...], ab0_ref[...] + ab1_ref[...]], axis=1)
    h = jnp.maximum(s1_ref[...] + agg, 0.0)
    s_ref[...] = jnp.dot(h, ws_ref[...],
                         preferred_element_type=jnp.float32) + b_ref[...]
    p_ref[...] = jnp.dot(h, wn_ref[...], preferred_element_type=jnp.float32)

  return pl.pallas_call(
      body,
      grid=(n // BLK,),
      in_specs=[
          pl.BlockSpec((BLK, din), lambda i: (i, 0)),
          pl.BlockSpec((BLK, HALF), lambda i: (i, 0)),
          pl.BlockSpec((BLK, HALF), lambda i: (i, 0)),
          pl.BlockSpec((BLK, HALF), lambda i: (i, 0)),
          pl.BlockSpec((BLK, HALF), lambda i: (i, 0)),
          pl.BlockSpec((din, dout), lambda i: (0, 0)),
          pl.BlockSpec((din, dout), lambda i: (0, 0)),
          pl.BlockSpec((1, dout), lambda i: (0, 0)),
      ],
      out_specs=[
          pl.BlockSpec((BLK, dout), lambda i: (i, 0)),
          pl.BlockSpec((BLK, dout), lambda i: (i, 0)),
      ],
      out_shape=[
          jax.ShapeDtypeStruct((n, dout), jnp.float32),
          jax.ShapeDtypeStruct((n, dout), jnp.float32),
      ],
  )(s1, aa0, aa1, ab0, ab1, ws, wn, b)


def _tc_add3(x, y, z):
  n, d = x.shape

  def body(x_ref, y_ref, z_ref, o_ref):
    o_ref[...] = x_ref[...] + y_ref[...] + z_ref[...]

  return pl.pallas_call(
      body,
      grid=(n // BLK,),
      in_specs=[pl.BlockSpec((BLK, d), lambda i: (i, 0))] * 3,
      out_specs=pl.BlockSpec((BLK, d), lambda i: (i, 0)),
      out_shape=jax.ShapeDtypeStruct((n, d), jnp.float32),
  )(x, y, z)


def kernel(inputs, edge_index, adj_high, W_self1, W_neigh1, b1,
           W_self2, W_neigh2, b2):
  src = edge_index[0]
  dst = edge_index[1]
  d_hid = W_self1.shape[1]
  n_cls = W_self2.shape[1]

  zrow = jnp.zeros((RCH, HALF), jnp.float32)
  zdeg = jnp.zeros((RCH, DEGW), jnp.float32)
  ones = jnp.ones((K, DEGW), jnp.float32)

  s1, p1 = _tc_proj2(inputs, W_self1, W_neigh1, b1.reshape(1, -1), d_hid)
  agg_a, agg_b, inv_deg = _make_seg1()(
      p1[:, :HALF], p1[:, HALF:], src, dst, zrow, zdeg, ones)

  s2, p2 = _tc_relu_proj2(s1, agg_a[:N_NODES], agg_a[N_NODES:],
                          agg_b[:N_NODES], agg_b[N_NODES:],
                          W_self2, W_neigh2, b2.reshape(1, -1), n_cls)
  agg2 = _make_seg2()(p2, src, dst, zrow, inv_deg)

  return _tc_add3(s2, agg2[:N_NODES], agg2[N_NODES:])
